# baseline (speedup 1.0000x reference)
import functools
import jax
import jax.numpy as jnp
from jax import lax
from jax.experimental import pallas as pl
from jax.experimental.pallas import tpu as pltpu
from jax.experimental.pallas import tpu_sc as plsc

D_MODEL = 768
B_TOTAL = 4 * 4096
NC, NS = 2, 16
NW = NC * NS
B_PER_W = B_TOTAL // NW
NBUF = 8
K = 16
NCHUNK = B_PER_W // K

_mesh = plsc.VectorSubcoreMesh(core_axis_name="c", subcore_axis_name="s")


@functools.partial(
    pl.kernel,
    mesh=_mesh,
    out_type=jax.ShapeDtypeStruct((B_TOTAL, D_MODEL), jnp.float32),
    scratch_types=[
        pltpu.VMEM((NCHUNK, K), jnp.int32),
        pltpu.VMEM((NBUF, K, D_MODEL), jnp.float32),
        pltpu.SemaphoreType.DMA((NBUF,)),
    ],
)
def _embed_sc(ids_hbm, table_hbm, out_hbm, idx_v, rows_v, sems):
    wid = lax.axis_index("s") * NC + lax.axis_index("c")
    base = wid * B_PER_W

    pltpu.sync_copy(ids_hbm.at[wid], idx_v)

    def gather(j, b):
        return pltpu.async_copy(
            table_hbm.at[idx_v.at[j]], rows_v.at[b], sems.at[b]
        )

    def store(j, b):
        return pltpu.async_copy(
            rows_v.at[b], out_hbm.at[pl.ds(base + j * K, K)], sems.at[b]
        )

    gh = [None] * NCHUNK
    sh = [None] * NCHUNK
    for b in range(NBUF):
        gh[b] = gather(b, b)
    for j in range(NCHUNK):
        b = j % NBUF
        gh[j].wait()
        sh[j] = store(j, b)
        jn = j + NBUF
        if jn < NCHUNK:
            sh[j].wait()
            gh[jn] = gather(jn, b)
    for j in range(NCHUNK - NBUF, NCHUNK):
        sh[j].wait()


def kernel(token_ids, weight):
    ids = token_ids.reshape(NW, NCHUNK, K).astype(jnp.int32)
    out = _embed_sc(ids, weight)
    return out.reshape(token_ids.shape + (D_MODEL,))


# speedup vs baseline: 1.0063x; 1.0063x over previous
import functools
import jax
import jax.numpy as jnp
from jax import lax
from jax.experimental import pallas as pl
from jax.experimental.pallas import tpu as pltpu
from jax.experimental.pallas import tpu_sc as plsc

D_MODEL = 768
B_TOTAL = 4 * 4096
NC, NS = 2, 16
NW = NC * NS
B_PER_W = B_TOTAL // NW
NBUF = 4
K = 32
NCHUNK = B_PER_W // K
DEFER = 1

_mesh = plsc.VectorSubcoreMesh(core_axis_name="c", subcore_axis_name="s")


@functools.partial(
    pl.kernel,
    mesh=_mesh,
    out_type=jax.ShapeDtypeStruct((B_TOTAL, D_MODEL), jnp.float32),
    scratch_types=[
        pltpu.VMEM((NCHUNK, K), jnp.int32),
        pltpu.VMEM((NBUF, K, D_MODEL), jnp.float32),
        pltpu.SemaphoreType.DMA((NBUF,)),
    ],
)
def _embed_sc(ids_hbm, table_hbm, out_hbm, idx_v, rows_v, sems):
    wid = lax.axis_index("s") * NC + lax.axis_index("c")
    base = wid * B_PER_W

    pltpu.sync_copy(ids_hbm.at[wid], idx_v)

    def gather(j):
        return pltpu.async_copy(
            table_hbm.at[idx_v.at[j]], rows_v.at[j % NBUF], sems.at[j % NBUF]
        )

    def store(j):
        return pltpu.async_copy(
            rows_v.at[j % NBUF], out_hbm.at[pl.ds(base + j * K, K)],
            sems.at[j % NBUF]
        )

    gh = [None] * NCHUNK
    sh = [None] * NCHUNK
    for b in range(min(NBUF, NCHUNK)):
        gh[b] = gather(b)
    for j in range(NCHUNK):
        gh[j].wait()
        sh[j] = store(j)
        jp = j - DEFER
        jn = jp + NBUF
        if jp >= 0 and jn < NCHUNK:
            sh[jp].wait()
            sh[jp] = None
            gh[jn] = gather(jn)
    for j in range(NCHUNK):
        if sh[j] is not None:
            sh[j].wait()


def kernel(token_ids, weight):
    ids = token_ids.reshape(NW, NCHUNK, K).astype(jnp.int32)
    out = _embed_sc(ids, weight)
    return out.reshape(token_ids.shape + (D_MODEL,))


# speedup vs baseline: 1.0126x; 1.0062x over previous
import functools
import jax
import jax.numpy as jnp
from jax import lax
from jax.experimental import pallas as pl
from jax.experimental.pallas import tpu as pltpu
from jax.experimental.pallas import tpu_sc as plsc

D_MODEL = 768
B_TOTAL = 4 * 4096
NC, NS = 2, 16
NW = NC * NS
B_PER_W = B_TOTAL // NW
NBUF = 5
K = 32
NCHUNK = B_PER_W // K
DEFER = 2

_mesh = plsc.VectorSubcoreMesh(core_axis_name="c", subcore_axis_name="s")


@functools.partial(
    pl.kernel,
    mesh=_mesh,
    out_type=jax.ShapeDtypeStruct((B_TOTAL, D_MODEL), jnp.float32),
    scratch_types=[
        pltpu.VMEM((NCHUNK, K), jnp.int32),
        pltpu.VMEM((NBUF, K, D_MODEL), jnp.float32),
        pltpu.SemaphoreType.DMA((NBUF,)),
    ],
)
def _embed_sc(ids_hbm, table_hbm, out_hbm, idx_v, rows_v, sems):
    wid = lax.axis_index("s") * NC + lax.axis_index("c")
    base = wid * B_PER_W

    pltpu.sync_copy(ids_hbm.at[wid], idx_v)

    def gather(j):
        return pltpu.async_copy(
            table_hbm.at[idx_v.at[j]], rows_v.at[j % NBUF], sems.at[j % NBUF]
        )

    def store(j):
        return pltpu.async_copy(
            rows_v.at[j % NBUF], out_hbm.at[pl.ds(base + j * K, K)],
            sems.at[j % NBUF]
        )

    gh = [None] * NCHUNK
    sh = [None] * NCHUNK
    for b in range(min(NBUF, NCHUNK)):
        gh[b] = gather(b)
    for j in range(NCHUNK):
        gh[j].wait()
        sh[j] = store(j)
        jp = j - DEFER
        jn = jp + NBUF
        if jp >= 0 and jn < NCHUNK:
            sh[jp].wait()
            sh[jp] = None
            gh[jn] = gather(jn)
    for j in range(NCHUNK):
        if sh[j] is not None:
            sh[j].wait()


def kernel(token_ids, weight):
    ids = token_ids.reshape(NW, NCHUNK, K).astype(jnp.int32)
    out = _embed_sc(ids, weight)
    return out.reshape(token_ids.shape + (D_MODEL,))


# speedup vs baseline: 1.0200x; 1.0073x over previous
import functools
import jax
import jax.numpy as jnp
from jax import lax
from jax.experimental import pallas as pl
from jax.experimental.pallas import tpu as pltpu
from jax.experimental.pallas import tpu_sc as plsc

D_MODEL = 768
B_TOTAL = 4 * 4096
NC, NS = 2, 16
NW = NC * NS
B_PER_W = B_TOTAL // NW
NBUF = 4
K = 32
NCHUNK = B_PER_W // K
DEFER = 0

_mesh = plsc.VectorSubcoreMesh(core_axis_name="c", subcore_axis_name="s")


@functools.partial(
    pl.kernel,
    mesh=_mesh,
    out_type=jax.ShapeDtypeStruct((B_TOTAL, D_MODEL), jnp.float32),
    scratch_types=[
        pltpu.VMEM((NCHUNK, K), jnp.int32),
        pltpu.VMEM((NBUF, K, D_MODEL), jnp.float32),
        pltpu.SemaphoreType.DMA((NBUF,)),
    ],
)
def _embed_sc(ids_hbm, table_hbm, out_hbm, idx_v, rows_v, sems):
    wid = lax.axis_index("s") * NC + lax.axis_index("c")
    base = wid * B_PER_W

    pltpu.sync_copy(ids_hbm.at[wid], idx_v)

    def gather(j):
        return pltpu.async_copy(
            table_hbm.at[idx_v.at[j]], rows_v.at[j % NBUF], sems.at[j % NBUF]
        )

    def store(j):
        return pltpu.async_copy(
            rows_v.at[j % NBUF], out_hbm.at[pl.ds(base + j * K, K)],
            sems.at[j % NBUF]
        )

    gh = [None] * NCHUNK
    sh = [None] * NCHUNK
    for b in range(min(NBUF, NCHUNK)):
        gh[b] = gather(b)
    for j in range(NCHUNK):
        gh[j].wait()
        sh[j] = store(j)
        jp = j - DEFER
        jn = jp + NBUF
        if jp >= 0 and jn < NCHUNK:
            sh[jp].wait()
            sh[jp] = None
            gh[jn] = gather(jn)
    for j in range(NCHUNK):
        if sh[j] is not None:
            sh[j].wait()


def kernel(token_ids, weight):
    ids = token_ids.reshape(NW, NCHUNK, K).astype(jnp.int32)
    out = _embed_sc(ids, weight)
    return out.reshape(token_ids.shape + (D_MODEL,))
